# baseline (speedup 1.0000x reference)
import jax
import jax.numpy as jnp
from jax import lax
from jax.experimental import pallas as pl
from jax.experimental.pallas import tpu as pltpu
from jax.experimental.pallas import tpu_sc as plsc

_B, _S, _D = 4, 2048, 1024
_NC = 2
_NW = 32
_S_PER_W = _S // _NW
_CROWS = 16
_SUBS = _S_PER_W // _CROWS
_NCHUNK = _SUBS * _B
_LANES = _D // 16


def _sc_body(x_hbm, pe_hbm, out_hbm,
             pe0, pe1, xb0, xb1,
             sem_pe0, sem_pe1, sem_in0, sem_in1, sem_out0, sem_out1):
    wid = lax.axis_index("s") * _NC + lax.axis_index("c")
    s0 = wid * _S_PER_W

    pe_bufs = (pe0, pe1)
    x_bufs = (xb0, xb1)
    pe_sems = (sem_pe0, sem_pe1)
    in_sems = (sem_in0, sem_in1)
    out_sems = (sem_out0, sem_out1)

    def x_row(k):
        sub, b = divmod(k, _B)
        return b * _S + s0 + sub * _CROWS

    def pe_row(sub):
        return s0 + sub * _CROWS

    pe_copies = {}
    in_copies = {}
    out_copies = {}

    pe_copies[0] = pltpu.async_copy(
        pe_hbm.at[pl.ds(pe_row(0), _CROWS), 0], pe0, sem_pe0)
    in_copies[0] = pltpu.async_copy(
        x_hbm.at[pl.ds(x_row(0), _CROWS)], xb0, sem_in0)

    for k in range(_NCHUNK):
        sub, b = divmod(k, _B)
        cur = k % 2
        pe_cur = sub % 2
        if b == 0:
            pe_copies[sub].wait()
            if sub + 1 < _SUBS:
                pe_copies[sub + 1] = pltpu.async_copy(
                    pe_hbm.at[pl.ds(pe_row(sub + 1), _CROWS), 0],
                    pe_bufs[(sub + 1) % 2], pe_sems[(sub + 1) % 2])
        if k + 1 < _NCHUNK:
            if k >= 1:
                out_copies[k - 1].wait()
            in_copies[k + 1] = pltpu.async_copy(
                x_hbm.at[pl.ds(x_row(k + 1), _CROWS)],
                x_bufs[(k + 1) % 2], in_sems[(k + 1) % 2])
        in_copies[k].wait()
        xr = x_bufs[cur]
        pr = pe_bufs[pe_cur]

        if b == 0:
            @plsc.parallel_loop(0, _CROWS * _D, 16, unroll=8)
            def _scale_add(i, xr=xr, pr=pr):
                r, c = i // _D, i % _D
                v = pr[r, pl.ds(c, 16)] * 0.002
                pr[r, pl.ds(c, 16)] = v
                plsc.addupdate(xr.at[r, pl.ds(c, 16)], v)
        else:
            @plsc.parallel_loop(0, _CROWS * _D, 16, unroll=16)
            def _add(i, xr=xr, pr=pr):
                r, c = i // _D, i % _D
                plsc.addupdate(xr.at[r, pl.ds(c, 16)], pr[r, pl.ds(c, 16)])

        out_copies[k] = pltpu.async_copy(
            xr, out_hbm.at[pl.ds(x_row(k), _CROWS)], out_sems[cur])

    out_copies[_NCHUNK - 2].wait()
    out_copies[_NCHUNK - 1].wait()


def kernel(x, pe):
    bz, lens, d = x.shape
    x2 = x.reshape(bz * lens, d)
    mesh = plsc.VectorSubcoreMesh(core_axis_name="c", subcore_axis_name="s")
    sc = pl.kernel(
        _sc_body,
        out_type=jax.ShapeDtypeStruct((bz * lens, d), x.dtype),
        mesh=mesh,
        compiler_params=pltpu.CompilerParams(
            use_tc_tiling_on_sc=True,
            skip_device_barrier=True,
            disable_semaphore_checks=True,
            disable_bounds_checks=True,
        ),
        scratch_types=[
            pltpu.VMEM((_CROWS, _D), jnp.float32),
            pltpu.VMEM((_CROWS, _D), jnp.float32),
            pltpu.VMEM((_CROWS, _D), jnp.float32),
            pltpu.VMEM((_CROWS, _D), jnp.float32),
            pltpu.SemaphoreType.DMA,
            pltpu.SemaphoreType.DMA,
            pltpu.SemaphoreType.DMA,
            pltpu.SemaphoreType.DMA,
            pltpu.SemaphoreType.DMA,
            pltpu.SemaphoreType.DMA,
        ],
    )
    return sc(x2, pe).reshape(bz, lens, d)


# speedup vs baseline: 1.0177x; 1.0177x over previous
import jax
import jax.numpy as jnp
from jax import lax
from jax.experimental import pallas as pl
from jax.experimental.pallas import tpu as pltpu
from jax.experimental.pallas import tpu_sc as plsc

_B, _S, _D = 4, 2048, 1024
_NC = 2
_NW = 32
_S_PER_W = _S // _NW
_CROWS = 32
_SUBS = _S_PER_W // _CROWS
_NCHUNK = _SUBS * _B


def _sc_body(x_hbm, pe_hbm, out_hbm,
             pe0, xb0, xb1,
             sem_pe, sem_in0, sem_in1, sem_out0, sem_out1):
    wid = lax.axis_index("s") * _NC + lax.axis_index("c")
    s0 = wid * _S_PER_W

    x_bufs = (xb0, xb1)
    in_sems = (sem_in0, sem_in1)
    out_sems = (sem_out0, sem_out1)

    def x_row(k):
        sub, b = divmod(k, _B)
        return b * _S + s0 + sub * _CROWS

    in_copies = {}
    out_copies = {}

    pe_copy = pltpu.async_copy(
        pe_hbm.at[pl.ds(s0, _CROWS), 0], pe0, sem_pe)
    in_copies[0] = pltpu.async_copy(
        x_hbm.at[pl.ds(x_row(0), _CROWS)], xb0, sem_in0)

    for k in range(_NCHUNK):
        sub, b = divmod(k, _B)
        cur = k % 2
        if b == 0:
            if sub > 0:
                pe_copy = pltpu.async_copy(
                    pe_hbm.at[pl.ds(s0 + sub * _CROWS, _CROWS), 0],
                    pe0, sem_pe)
            pe_copy.wait()
        if k + 1 < _NCHUNK:
            if k >= 1:
                out_copies[k - 1].wait()
            in_copies[k + 1] = pltpu.async_copy(
                x_hbm.at[pl.ds(x_row(k + 1), _CROWS)],
                x_bufs[(k + 1) % 2], in_sems[(k + 1) % 2])
        in_copies[k].wait()
        xr = x_bufs[cur]

        if b == 0:
            @plsc.parallel_loop(0, _CROWS * _D, 16, unroll=8)
            def _scale_add(i, xr=xr, pr=pe0):
                r, c = i // _D, i % _D
                v = pr[r, pl.ds(c, 16)] * 0.002
                pr[r, pl.ds(c, 16)] = v
                plsc.addupdate(xr.at[r, pl.ds(c, 16)], v)
        else:
            @plsc.parallel_loop(0, _CROWS * _D, 16, unroll=16)
            def _add(i, xr=xr, pr=pe0):
                r, c = i // _D, i % _D
                plsc.addupdate(xr.at[r, pl.ds(c, 16)], pr[r, pl.ds(c, 16)])

        out_copies[k] = pltpu.async_copy(
            xr, out_hbm.at[pl.ds(x_row(k), _CROWS)], out_sems[cur])

    out_copies[_NCHUNK - 2].wait()
    out_copies[_NCHUNK - 1].wait()


def kernel(x, pe):
    bz, lens, d = x.shape
    x2 = x.reshape(bz * lens, d)
    mesh = plsc.VectorSubcoreMesh(core_axis_name="c", subcore_axis_name="s")
    sc = pl.kernel(
        _sc_body,
        out_type=jax.ShapeDtypeStruct((bz * lens, d), x.dtype),
        mesh=mesh,
        compiler_params=pltpu.CompilerParams(
            use_tc_tiling_on_sc=True,
            skip_device_barrier=True,
            disable_semaphore_checks=True,
            disable_bounds_checks=True,
        ),
        scratch_types=[
            pltpu.VMEM((_CROWS, _D), jnp.float32),
            pltpu.VMEM((_CROWS, _D), jnp.float32),
            pltpu.VMEM((_CROWS, _D), jnp.float32),
            pltpu.SemaphoreType.DMA,
            pltpu.SemaphoreType.DMA,
            pltpu.SemaphoreType.DMA,
            pltpu.SemaphoreType.DMA,
            pltpu.SemaphoreType.DMA,
        ],
    )
    return sc(x2, pe).reshape(bz, lens, d)


# speedup vs baseline: 1.0502x; 1.0320x over previous
import jax
import jax.numpy as jnp
from jax import lax
from jax.experimental import pallas as pl
from jax.experimental.pallas import tpu as pltpu
from jax.experimental.pallas import tpu_sc as plsc

_B, _S, _D = 4, 2048, 1024
_NC = 2
_NW = 32
_S_PER_W = _S // _NW
_CROWS = 16
_SUBS = _S_PER_W // _CROWS
_NCHUNK = _SUBS * _B


def _sc_body(x_hbm, pe_hbm, out_hbm,
             pe0, pe1, xb0, xb1, xb2,
             sem_pe0, sem_pe1,
             sem_in0, sem_in1, sem_in2,
             sem_out0, sem_out1, sem_out2):
    wid = lax.axis_index("s") * _NC + lax.axis_index("c")
    s0 = wid * _S_PER_W

    pe_bufs = (pe0, pe1)
    pe_sems = (sem_pe0, sem_pe1)
    x_bufs = (xb0, xb1, xb2)
    in_sems = (sem_in0, sem_in1, sem_in2)
    out_sems = (sem_out0, sem_out1, sem_out2)

    def x_row(k):
        sub, b = divmod(k, _B)
        return b * _S + s0 + sub * _CROWS

    def pe_row(sub):
        return s0 + sub * _CROWS

    pe_copies = {}
    in_copies = {}
    out_copies = {}

    pe_copies[0] = pltpu.async_copy(
        pe_hbm.at[pl.ds(pe_row(0), _CROWS), 0], pe0, sem_pe0)
    in_copies[0] = pltpu.async_copy(
        x_hbm.at[pl.ds(x_row(0), _CROWS)], xb0, sem_in0)
    in_copies[1] = pltpu.async_copy(
        x_hbm.at[pl.ds(x_row(1), _CROWS)], xb1, sem_in1)

    for k in range(_NCHUNK):
        sub, b = divmod(k, _B)
        if b == 0:
            pe_copies[sub].wait()
            if sub + 1 < _SUBS:
                pe_copies[sub + 1] = pltpu.async_copy(
                    pe_hbm.at[pl.ds(pe_row(sub + 1), _CROWS), 0],
                    pe_bufs[(sub + 1) % 2], pe_sems[(sub + 1) % 2])
        in_copies[k].wait()
        xr = x_bufs[k % 3]
        pr = pe_bufs[sub % 2]

        if b == 0:
            @plsc.parallel_loop(0, _CROWS * _D, 16, unroll=8)
            def _scale_add(i, xr=xr, pr=pr):
                r, c = i // _D, i % _D
                v = pr[r, pl.ds(c, 16)] * 0.002
                pr[r, pl.ds(c, 16)] = v
                plsc.addupdate(xr.at[r, pl.ds(c, 16)], v)
        else:
            @plsc.parallel_loop(0, _CROWS * _D, 16, unroll=16)
            def _add(i, xr=xr, pr=pr):
                r, c = i // _D, i % _D
                plsc.addupdate(xr.at[r, pl.ds(c, 16)], pr[r, pl.ds(c, 16)])

        out_copies[k] = pltpu.async_copy(
            xr, out_hbm.at[pl.ds(x_row(k), _CROWS)], out_sems[k % 3])
        if k + 2 < _NCHUNK:
            if k >= 1:
                out_copies[k - 1].wait()
            in_copies[k + 2] = pltpu.async_copy(
                x_hbm.at[pl.ds(x_row(k + 2), _CROWS)],
                x_bufs[(k + 2) % 3], in_sems[(k + 2) % 3])

    out_copies[_NCHUNK - 3].wait()
    out_copies[_NCHUNK - 2].wait()
    out_copies[_NCHUNK - 1].wait()


def kernel(x, pe):
    bz, lens, d = x.shape
    x2 = x.reshape(bz * lens, d)
    mesh = plsc.VectorSubcoreMesh(core_axis_name="c", subcore_axis_name="s")
    sc = pl.kernel(
        _sc_body,
        out_type=jax.ShapeDtypeStruct((bz * lens, d), x.dtype),
        mesh=mesh,
        compiler_params=pltpu.CompilerParams(
            use_tc_tiling_on_sc=True,
            skip_device_barrier=True,
            disable_semaphore_checks=True,
            disable_bounds_checks=True,
        ),
        scratch_types=[
            pltpu.VMEM((_CROWS, _D), jnp.float32),
            pltpu.VMEM((_CROWS, _D), jnp.float32),
            pltpu.VMEM((_CROWS, _D), jnp.float32),
            pltpu.VMEM((_CROWS, _D), jnp.float32),
            pltpu.VMEM((_CROWS, _D), jnp.float32),
            pltpu.SemaphoreType.DMA,
            pltpu.SemaphoreType.DMA,
            pltpu.SemaphoreType.DMA,
            pltpu.SemaphoreType.DMA,
            pltpu.SemaphoreType.DMA,
            pltpu.SemaphoreType.DMA,
            pltpu.SemaphoreType.DMA,
            pltpu.SemaphoreType.DMA,
        ],
    )
    return sc(x2, pe).reshape(bz, lens, d)
